# Initial kernel scaffold; baseline (speedup 1.0000x reference)
import functools

import jax
import jax.numpy as jnp
from jax import lax
from jax.experimental import pallas as pl
from jax.experimental.pallas import tpu as pltpu
from jax.experimental.pallas import tpu_sc as plsc

N = 10000
E = 320000
D = 128

NC = 2
NS = 16
NW = NC * NS
EPT = E // NW
K = 125
C = EPT // K
RPT = N // NS


def _make_agg(with_deg: bool):
    mesh = plsc.VectorSubcoreMesh(core_axis_name="c", subcore_axis_name="s")
    out_type = [jax.ShapeDtypeStruct((NC, N, D), jnp.float32)]
    if with_deg:
        out_type.append(jax.ShapeDtypeStruct((NC, N, 16), jnp.float32))
    scratch = [
        pltpu.VMEM_SHARED((N, D), jnp.float32),
        pltpu.VMEM((C, K), jnp.int32),
        pltpu.VMEM((C, K), jnp.int32),
        pltpu.VMEM((K, D), jnp.float32),
        pltpu.VMEM((K, D), jnp.float32),
        pltpu.SemaphoreType.DMA,
        pltpu.SemaphoreType.DMA,
    ]
    if with_deg:
        scratch += [
            pltpu.VMEM_SHARED((N, 16), jnp.float32),
            pltpu.VMEM((K, 16), jnp.float32),
        ]

    def body(h_hbm, src_hbm, dst_hbm, zeros_hbm, ones_hbm, *rest):
        if with_deg:
            (out_hbm, deg_hbm, acc, srcv, dstv, bufa, bufb, sema, semb,
             dacc, ones_v) = rest
        else:
            out_hbm, acc, srcv, dstv, bufa, bufb, sema, semb = rest
        cid = lax.axis_index("c")
        sid = lax.axis_index("s")
        wid = sid * NC + cid
        row0 = sid * RPT

        pltpu.sync_copy(zeros_hbm.at[pl.ds(row0, RPT)],
                        acc.at[pl.ds(row0, RPT)])
        if with_deg:
            pltpu.sync_copy(zeros_hbm.at[pl.ds(row0, RPT), pl.ds(0, 16)],
                            dacc.at[pl.ds(row0, RPT)])
            pltpu.sync_copy(ones_hbm, ones_v)
        pltpu.sync_copy(src_hbm.at[wid], srcv)
        pltpu.sync_copy(dst_hbm.at[wid], dstv)
        plsc.subcore_barrier()

        def process(c, buf):
            pltpu.sync_copy(buf, acc.at[dstv.at[c]], add=True)
            if with_deg:
                pltpu.sync_copy(ones_v, dacc.at[dstv.at[c]], add=True)

        pltpu.async_copy(h_hbm.at[srcv.at[0]], bufa, sema)

        @pl.loop(0, C, step=2)
        def _(c):
            pltpu.async_copy(h_hbm.at[srcv.at[c + 1]], bufb, semb)
            pltpu.make_async_copy(h_hbm.at[srcv.at[c]], bufa, sema).wait()
            process(c, bufa)

            @pl.when(c + 2 < C)
            def _():
                pltpu.async_copy(h_hbm.at[srcv.at[c + 2]], bufa, sema)

            pltpu.make_async_copy(h_hbm.at[srcv.at[c + 1]], bufb, semb).wait()
            process(c + 1, bufb)

        plsc.subcore_barrier()
        pltpu.sync_copy(acc.at[pl.ds(row0, RPT)],
                        out_hbm.at[cid, pl.ds(row0, RPT)])
        if with_deg:
            pltpu.sync_copy(dacc.at[pl.ds(row0, RPT)],
                            deg_hbm.at[cid, pl.ds(row0, RPT)])

    return pl.kernel(body, out_type=tuple(out_type) if with_deg else out_type[0],
                     mesh=mesh, scratch_types=scratch)


_agg_deg = _make_agg(True)
_agg = _make_agg(False)


def _tc_layer(p, pdeg, h, W_l, W_r, b, gamma, beta, W_h=None, b_h=None):
    B = 2000
    final = W_h is not None

    def body(*refs):
        if final:
            (p_ref, pd_ref, h_ref, wl_ref, wr_ref, b_ref, g_ref, be_ref,
             wh_ref, bh_ref, o_ref) = refs
        else:
            (p_ref, pd_ref, h_ref, wl_ref, wr_ref, b_ref, g_ref, be_ref,
             o_ref) = refs
        deg = pd_ref[0, :, 0:1] + pd_ref[1, :, 0:1]
        deg = jnp.maximum(deg, 1.0)
        agg = (p_ref[0] + p_ref[1]) / deg
        z = (jnp.dot(agg, wl_ref[...], preferred_element_type=jnp.float32)
             + jnp.dot(h_ref[...], wr_ref[...], preferred_element_type=jnp.float32)
             + b_ref[...])
        mu = jnp.mean(z, axis=-1, keepdims=True)
        zc = z - mu
        var = jnp.mean(zc * zc, axis=-1, keepdims=True)
        z = g_ref[...] * zc / jnp.sqrt(var + 1e-5) + be_ref[...]
        z = jnp.maximum(z, 0.0)
        if final:
            z = (jnp.dot(z, wh_ref[...], preferred_element_type=jnp.float32)
                 + bh_ref[...])
        o_ref[...] = z

    in_specs = [
        pl.BlockSpec((NC, B, D), lambda i: (0, i, 0)),
        pl.BlockSpec((NC, B, 16), lambda i: (0, i, 0)),
        pl.BlockSpec((B, D), lambda i: (i, 0)),
        pl.BlockSpec((D, D), lambda i: (0, 0)),
        pl.BlockSpec((D, D), lambda i: (0, 0)),
        pl.BlockSpec((D,), lambda i: (0,)),
        pl.BlockSpec((D,), lambda i: (0,)),
        pl.BlockSpec((D,), lambda i: (0,)),
    ]
    args = [p, pdeg, h, W_l, W_r, b, gamma, beta]
    if final:
        in_specs += [pl.BlockSpec((D, D), lambda i: (0, 0)),
                     pl.BlockSpec((D,), lambda i: (0,))]
        args += [W_h, b_h]
    return pl.pallas_call(
        body,
        grid=(N // B,),
        in_specs=in_specs,
        out_specs=pl.BlockSpec((B, D), lambda i: (i, 0)),
        out_shape=jax.ShapeDtypeStruct((N, D), jnp.float32),
    )(*args)


def kernel(x, edge_index, W_l0, W_r0, b0, gamma0, beta0,
           W_l1, W_r1, b1, gamma1, beta1, W_h, b_h):
    src = edge_index[0].astype(jnp.int32).reshape(NW, C, K)
    dst = edge_index[1].astype(jnp.int32).reshape(NW, C, K)
    zeros = jnp.zeros((N, D), jnp.float32)
    ones = jnp.ones((K, 16), jnp.float32)

    p0, pdeg = _agg_deg(x, src, dst, zeros, ones)
    h1 = _tc_layer(p0, pdeg, x, W_l0, W_r0, b0, gamma0, beta0)
    p1 = _agg(h1, src, dst, zeros, ones)
    out = _tc_layer(p1, pdeg, h1, W_l1, W_r1, b1, gamma1, beta1, W_h, b_h)
    return out


# speedup vs baseline: 3.7880x; 3.7880x over previous
import functools

import jax
import jax.numpy as jnp
from jax import lax
from jax.experimental import pallas as pl
from jax.experimental.pallas import tpu as pltpu
from jax.experimental.pallas import tpu_sc as plsc

N = 10000
E = 320000
D = 128

NC = 2
NS = 16
NW = NC * NS
K = 128
EPT = 10240
G = EPT // (4 * K)
NPAD = 10112
RPT = NPAD // NS


def _make_agg():
    mesh = plsc.VectorSubcoreMesh(core_axis_name="c", subcore_axis_name="s",
                                  num_cores=NC, num_subcores=NS)
    scratch = [
        pltpu.VMEM_SHARED((NPAD, D), jnp.float32),
        pltpu.VMEM((8, K), jnp.int32),
        pltpu.VMEM((8, K), jnp.int32),
        pltpu.VMEM((K, D), jnp.float32),
        pltpu.VMEM((K, D), jnp.float32),
        pltpu.SemaphoreType.DMA,
        pltpu.SemaphoreType.DMA,
        pltpu.SemaphoreType.DMA,
    ]

    def body(h_hbm, idx_hbm, zeros_hbm, out_hbm,
             acc, iba, ibb, rba, rbb, isem, sema, semb):
        cid = lax.axis_index("c")
        sid = lax.axis_index("s")
        wid = sid * NC + cid
        row0 = sid * RPT

        pltpu.sync_copy(zeros_hbm.at[pl.ds(row0, RPT)],
                        acc.at[pl.ds(row0, RPT)])
        plsc.subcore_barrier()

        @pl.loop(0, G)
        def _(g):
            pltpu.async_copy(idx_hbm.at[wid, g], iba, isem).wait()
            for j in range(4):
                pltpu.async_copy(h_hbm.at[iba.at[j]], rba, sema).wait()
                pltpu.sync_copy(rba, acc.at[iba.at[4 + j]], add=True)

        plsc.subcore_barrier()
        pltpu.sync_copy(acc.at[pl.ds(row0, RPT)],
                        out_hbm.at[cid, pl.ds(row0, RPT)])

    return pl.kernel(body,
                     out_type=jax.ShapeDtypeStruct((NC, NPAD, D), jnp.float32),
                     mesh=mesh, scratch_types=scratch)


def _make_deg():
    import dataclasses
    mesh = plsc.VectorSubcoreMesh(core_axis_name="c", subcore_axis_name="s",
                                  num_cores=NC, num_subcores=NS)
    cp = pltpu.CompilerParams()
    if "needs_layout_passes" in pltpu.CompilerParams.__dataclass_fields__:
        cp = dataclasses.replace(cp, needs_layout_passes=False)
    scratch = [
        pltpu.VMEM((NPAD,), jnp.float32),
        pltpu.VMEM((8, K), jnp.int32),
        pltpu.SemaphoreType.DMA,
    ]

    def body(idx_hbm, deg_hbm, hist, iba, isem):
        cid = lax.axis_index("c")
        sid = lax.axis_index("s")
        wid = sid * NC + cid

        @pl.loop(0, NPAD // 16)
        def _(i):
            hist[pl.ds(i * 16, 16)] = jnp.zeros((16,), jnp.float32)

        ones16 = jnp.ones((16,), jnp.float32)

        @pl.loop(0, G)
        def _(g):
            pltpu.async_copy(idx_hbm.at[wid, g], iba, isem).wait()
            for j in range(4):
                for l in range(K // 16):
                    ids = iba[4 + j, pl.ds(l * 16, 16)]
                    plsc.addupdate_scatter(hist, [ids], ones16)

        pltpu.sync_copy(hist, deg_hbm.at[wid])

    return pl.kernel(body,
                     out_type=jax.ShapeDtypeStruct((NW, NPAD), jnp.float32),
                     mesh=mesh, compiler_params=cp, scratch_types=scratch)


_make_agg = functools.cache(_make_agg)
_make_deg = functools.cache(_make_deg)


def _tc_layer(p, pdeg, h, W_l, W_r, b, gamma, beta, W_h=None, b_h=None):
    final = W_h is not None

    def body(*refs):
        if final:
            (p_ref, pd_ref, h_ref, wl_ref, wr_ref, b_ref, g_ref, be_ref,
             wh_ref, bh_ref, o_ref) = refs
        else:
            (p_ref, pd_ref, h_ref, wl_ref, wr_ref, b_ref, g_ref, be_ref,
             o_ref) = refs
        deg = lax.dot_general(pd_ref[...], jnp.ones((NW, 1), jnp.float32),
                              (((0,), (0,)), ((), ())),
                              preferred_element_type=jnp.float32)
        deg = jnp.maximum(deg[:N], 1.0)
        agg = (p_ref[0, :N, :] + p_ref[1, :N, :]) / deg
        z = (jnp.dot(agg, wl_ref[...], preferred_element_type=jnp.float32)
             + jnp.dot(h_ref[...], wr_ref[...], preferred_element_type=jnp.float32)
             + b_ref[...])
        mu = jnp.mean(z, axis=-1, keepdims=True)
        zc = z - mu
        var = jnp.mean(zc * zc, axis=-1, keepdims=True)
        z = g_ref[...] * zc / jnp.sqrt(var + 1e-5) + be_ref[...]
        z = jnp.maximum(z, 0.0)
        if final:
            z = (jnp.dot(z, wh_ref[...], preferred_element_type=jnp.float32)
                 + bh_ref[...])
        o_ref[...] = z

    args = [p, pdeg, h, W_l, W_r, b, gamma, beta]
    if final:
        args += [W_h, b_h]
    return pl.pallas_call(
        body,
        out_shape=jax.ShapeDtypeStruct((N, D), jnp.float32),
    )(*args)


def kernel(x, edge_index, W_l0, W_r0, b0, gamma0, beta0,
           W_l1, W_r1, b1, gamma1, beta1, W_h, b_h):
    src = edge_index[0].astype(jnp.int32).reshape(NW, E // NW)
    dst = edge_index[1].astype(jnp.int32).reshape(NW, E // NW)
    pad = EPT - E // NW
    src = jnp.pad(src, ((0, 0), (0, pad)))
    dst = jnp.pad(dst, ((0, 0), (0, pad)), constant_values=N)
    packed = jnp.concatenate([src.reshape(NW, G, 4, K),
                              dst.reshape(NW, G, 4, K)], axis=2)
    zeros = jnp.zeros((NPAD, D), jnp.float32)

    pdeg = _make_deg()(packed)
    p0 = _make_agg()(x, packed, zeros)
    h1 = _tc_layer(p0, pdeg, x, W_l0, W_r0, b0, gamma0, beta0)
    p1 = _make_agg()(h1, packed, zeros)
    out = _tc_layer(p1, pdeg, h1, W_l1, W_r1, b1, gamma1, beta1, W_h, b_h)
    return out


# speedup vs baseline: 4.0545x; 1.0703x over previous
import functools

import jax
import jax.numpy as jnp
from jax import lax
from jax.experimental import pallas as pl
from jax.experimental.pallas import tpu as pltpu
from jax.experimental.pallas import tpu_sc as plsc

N = 10000
E = 320000
D = 128

NC = 2
NS = 16
NW = NC * NS
K = 128
EPT = 10240
G = EPT // (4 * K)
NPAD = 10112
RPT = NPAD // NS


def _make_agg():
    mesh = plsc.VectorSubcoreMesh(core_axis_name="c", subcore_axis_name="s",
                                  num_cores=NC, num_subcores=NS)
    GH = G // 2
    scratch = [
        pltpu.VMEM_SHARED((NPAD, D), jnp.float32),
        pltpu.VMEM((GH, 8, K), jnp.int32),
        pltpu.VMEM((K, D), jnp.float32),
        pltpu.VMEM((K, D), jnp.float32),
        pltpu.SemaphoreType.DMA,
        pltpu.SemaphoreType.DMA,
        pltpu.SemaphoreType.DMA,
        pltpu.SemaphoreType.DMA,
    ]

    def body(h_hbm, idx_hbm, zeros_hbm, out_hbm,
             acc, idxv, rb0, rb1, g0, g1, s0, s1):
        cid = lax.axis_index("c")
        sid = lax.axis_index("s")
        wid = sid * NC + cid
        row0 = sid * RPT

        pltpu.sync_copy(zeros_hbm.at[pl.ds(row0, RPT)],
                        acc.at[pl.ds(row0, RPT)])
        plsc.subcore_barrier()

        rbufs = (rb0, rb1)
        gsems = (g0, g1)
        ssems = (s0, s1)

        for half in range(2):
            pltpu.sync_copy(idx_hbm.at[wid, pl.ds(half * GH, GH)], idxv)

            @pl.loop(0, GH)
            def _(g):
                for pair in range(2):
                    ga = pltpu.async_copy(
                        h_hbm.at[idxv.at[g, 2 * pair]], rb0, g0)
                    gb = pltpu.async_copy(
                        h_hbm.at[idxv.at[g, 2 * pair + 1]], rb1, g1)
                    ga.wait()
                    sa = pltpu.async_copy(
                        rb0, acc.at[idxv.at[g, 4 + 2 * pair]], s0, add=True)
                    gb.wait()
                    sb = pltpu.async_copy(
                        rb1, acc.at[idxv.at[g, 4 + 2 * pair + 1]], s1,
                        add=True)
                    sa.wait()
                    sb.wait()

        plsc.subcore_barrier()
        pltpu.sync_copy(acc.at[pl.ds(row0, RPT)],
                        out_hbm.at[cid, pl.ds(row0, RPT)])

    return pl.kernel(body,
                     out_type=jax.ShapeDtypeStruct((NC, NPAD, D), jnp.float32),
                     mesh=mesh, scratch_types=scratch)


def _make_deg():
    import dataclasses
    mesh = plsc.VectorSubcoreMesh(core_axis_name="c", subcore_axis_name="s",
                                  num_cores=NC, num_subcores=NS)
    cp = pltpu.CompilerParams()
    if "needs_layout_passes" in pltpu.CompilerParams.__dataclass_fields__:
        cp = dataclasses.replace(cp, needs_layout_passes=False)
    scratch = [
        pltpu.VMEM((NPAD,), jnp.float32),
        pltpu.VMEM((8, K), jnp.int32),
        pltpu.SemaphoreType.DMA,
    ]

    def body(idx_hbm, deg_hbm, hist, iba, isem):
        cid = lax.axis_index("c")
        sid = lax.axis_index("s")
        wid = sid * NC + cid

        @pl.loop(0, NPAD // 16)
        def _(i):
            hist[pl.ds(i * 16, 16)] = jnp.zeros((16,), jnp.float32)

        ones16 = jnp.ones((16,), jnp.float32)

        @pl.loop(0, G)
        def _(g):
            pltpu.async_copy(idx_hbm.at[wid, g], iba, isem).wait()
            for j in range(4):
                for l in range(K // 16):
                    ids = iba[4 + j, pl.ds(l * 16, 16)]
                    plsc.addupdate_scatter(hist, [ids], ones16)

        pltpu.sync_copy(hist, deg_hbm.at[wid])

    return pl.kernel(body,
                     out_type=jax.ShapeDtypeStruct((NW, NPAD), jnp.float32),
                     mesh=mesh, compiler_params=cp, scratch_types=scratch)


_make_agg = functools.cache(_make_agg)
_make_deg = functools.cache(_make_deg)


def _tc_layer(p, pdeg, h, W_l, W_r, b, gamma, beta, W_h=None, b_h=None):
    final = W_h is not None

    def body(*refs):
        if final:
            (p_ref, pd_ref, h_ref, wl_ref, wr_ref, b_ref, g_ref, be_ref,
             wh_ref, bh_ref, o_ref) = refs
        else:
            (p_ref, pd_ref, h_ref, wl_ref, wr_ref, b_ref, g_ref, be_ref,
             o_ref) = refs
        deg = lax.dot_general(pd_ref[...], jnp.ones((NW, 1), jnp.float32),
                              (((0,), (0,)), ((), ())),
                              preferred_element_type=jnp.float32)
        deg = jnp.maximum(deg[:N], 1.0)
        agg = (p_ref[0, :N, :] + p_ref[1, :N, :]) / deg
        z = (jnp.dot(agg, wl_ref[...], preferred_element_type=jnp.float32)
             + jnp.dot(h_ref[...], wr_ref[...], preferred_element_type=jnp.float32)
             + b_ref[...])
        mu = jnp.mean(z, axis=-1, keepdims=True)
        zc = z - mu
        var = jnp.mean(zc * zc, axis=-1, keepdims=True)
        z = g_ref[...] * zc / jnp.sqrt(var + 1e-5) + be_ref[...]
        z = jnp.maximum(z, 0.0)
        if final:
            z = (jnp.dot(z, wh_ref[...], preferred_element_type=jnp.float32)
                 + bh_ref[...])
        o_ref[...] = z

    args = [p, pdeg, h, W_l, W_r, b, gamma, beta]
    if final:
        args += [W_h, b_h]
    return pl.pallas_call(
        body,
        out_shape=jax.ShapeDtypeStruct((N, D), jnp.float32),
    )(*args)


def kernel(x, edge_index, W_l0, W_r0, b0, gamma0, beta0,
           W_l1, W_r1, b1, gamma1, beta1, W_h, b_h):
    src = edge_index[0].astype(jnp.int32).reshape(NW, E // NW)
    dst = edge_index[1].astype(jnp.int32).reshape(NW, E // NW)
    pad = EPT - E // NW
    src = jnp.pad(src, ((0, 0), (0, pad)))
    dst = jnp.pad(dst, ((0, 0), (0, pad)), constant_values=N)
    packed = jnp.concatenate([src.reshape(NW, G, 4, K),
                              dst.reshape(NW, G, 4, K)], axis=2)
    zeros = jnp.zeros((NPAD, D), jnp.float32)

    pdeg = _make_deg()(packed)
    p0 = _make_agg()(x, packed, zeros)
    h1 = _tc_layer(p0, pdeg, x, W_l0, W_r0, b0, gamma0, beta0)
    p1 = _make_agg()(h1, packed, zeros)
    out = _tc_layer(p1, pdeg, h1, W_l1, W_r1, b1, gamma1, beta1, W_h, b_h)
    return out
